# baseline (speedup 1.0000x reference)
import functools

import jax
import jax.numpy as jnp
from jax import lax
from jax.experimental import pallas as pl
from jax.experimental.pallas import tpu as pltpu
from jax.experimental.pallas import tpu_sc as plsc

B = 16384
D = 32
L = 50
LANES = 128

_info = plsc.get_sparse_core_info()
_NC = _info.num_cores
_NS = _info.num_subcores
NW = _NC * _NS
B_PER_W = B // NW
NSLOT = 8
GVEC = 16
GROUPS = B_PER_W // GVEC

_mesh = plsc.VectorSubcoreMesh(core_axis_name="c", subcore_axis_name="s")


@functools.partial(
    pl.kernel,
    mesh=_mesh,
    out_type=jax.ShapeDtypeStruct((B, D), jnp.float32),
    scratch_types=[
        pltpu.VMEM((2, B_PER_W), jnp.int32),
        pltpu.VMEM((NSLOT, D, LANES), jnp.float32),
        pltpu.VMEM((B_PER_W, D), jnp.float32),
        pltpu.SemaphoreType.DMA((NSLOT,)),
    ],
    compiler_params=pltpu.CompilerParams(needs_layout_passes=False),
)
def _gather_kernel(act_t_hbm, table_t_hbm, out_hbm, idx_v, stage_v, out_v, sems):
    wid = lax.axis_index("s") * _NC + lax.axis_index("c")
    base = wid * B_PER_W
    pltpu.sync_copy(act_t_hbm.at[pl.ds(48, 2), pl.ds(base, B_PER_W)], idx_v)

    rows_lo = lax.iota(jnp.int32, 16)
    rows_hi = rows_lo + 16

    def fire(v, slot):
        col0 = pl.multiple_of((v >> 7) * LANES, LANES)
        pltpu.async_copy(
            table_t_hbm.at[:, pl.ds(col0, LANES)],
            stage_v.at[slot],
            sems.at[slot],
        )

    def vec16(g):
        return idx_v[1, pl.ds(g * GVEC, GVEC)]

    vec0 = vec16(0)
    for k in range(NSLOT):
        fire(vec0[k], k)

    def extract(vec, k, slot, j):
        pltpu.make_async_copy(
            table_t_hbm.at[:, pl.ds(0, LANES)],
            stage_v.at[slot],
            sems.at[slot],
        ).wait()
        lane = jnp.full((16,), vec[k] & 127, jnp.int32)
        lo = plsc.load_gather(stage_v.at[slot], [rows_lo, lane])
        hi = plsc.load_gather(stage_v.at[slot], [rows_hi, lane])
        out_v[j, pl.ds(0, 16)] = lo
        out_v[j, pl.ds(16, 16)] = hi

    def body(g, carry):
        vec = vec16(g)
        vec_next = vec16(g + 1)
        for k in range(GVEC):
            slot = k % NSLOT
            j = g * GVEC + k
            extract(vec, k, slot, j)
            if k < NSLOT:
                fire(vec[k + NSLOT], slot)
            else:
                fire(vec_next[k - NSLOT], slot)
        return carry

    lax.fori_loop(0, GROUPS - 1, body, 0)
    vec = vec16(GROUPS - 1)
    for k in range(GVEC):
        slot = k % NSLOT
        j = (GROUPS - 1) * GVEC + k
        extract(vec, k, slot, j)
        if k < NSLOT:
            fire(vec[k + NSLOT], slot)
    pltpu.sync_copy(out_v, out_hbm.at[pl.ds(base, B_PER_W)])


def kernel(actionList, table):
    return _gather_kernel(actionList.astype(jnp.int32).T, table.T)


# speedup vs baseline: 1.0075x; 1.0075x over previous
import functools

import jax
import jax.numpy as jnp
from jax import lax
from jax.experimental import pallas as pl
from jax.experimental.pallas import tpu as pltpu
from jax.experimental.pallas import tpu_sc as plsc

B = 16384
D = 32
L = 50
LANES = 128

_info = plsc.get_sparse_core_info()
_NC = _info.num_cores
_NS = _info.num_subcores
NW = _NC * _NS
B_PER_W = B // NW
NSLOT = 8
GVEC = 16
GROUPS = B_PER_W // GVEC

_mesh = plsc.VectorSubcoreMesh(core_axis_name="c", subcore_axis_name="s")


@functools.partial(
    pl.kernel,
    mesh=_mesh,
    out_type=jax.ShapeDtypeStruct((B, D), jnp.float32),
    scratch_types=[
        pltpu.VMEM((2, B_PER_W), jnp.int32),
        pltpu.VMEM((NSLOT, D, LANES), jnp.float32),
        pltpu.VMEM((B_PER_W, D), jnp.float32),
        pltpu.SemaphoreType.DMA((NSLOT,)),
    ],
    compiler_params=pltpu.CompilerParams(needs_layout_passes=False),
)
def _gather_kernel(act_t_hbm, table_t_hbm, out_hbm, idx_v, stage_v, out_v, sems):
    wid = lax.axis_index("s") * _NC + lax.axis_index("c")
    base = wid * B_PER_W
    pltpu.sync_copy(act_t_hbm.at[pl.ds(48, 2), pl.ds(base, B_PER_W)], idx_v)

    rows_lo = lax.iota(jnp.int32, 16)
    rows_hi = rows_lo + 16

    def fire(v, slot):
        col0 = pl.multiple_of((v >> 7) * LANES, LANES)
        pltpu.async_copy(
            table_t_hbm.at[:, pl.ds(col0, LANES)],
            stage_v.at[slot],
            sems.at[slot],
        )

    def vec16(g):
        return idx_v[1, pl.ds(g * GVEC, GVEC)]

    vec0 = vec16(0)
    for k in range(NSLOT):
        fire(vec0[k], k)

    def body(g, carry):
        vec = vec16(g)
        vec_next = vec16((g + 1) % GROUPS)
        last = g == GROUPS - 1
        for k in range(GVEC):
            slot = k % NSLOT
            pltpu.make_async_copy(
                table_t_hbm.at[:, pl.ds(0, LANES)],
                stage_v.at[slot],
                sems.at[slot],
            ).wait()
            lane = jnp.full((16,), vec[k] & 127, jnp.int32)
            lo = plsc.load_gather(stage_v.at[slot], [rows_lo, lane])
            hi = plsc.load_gather(stage_v.at[slot], [rows_hi, lane])
            j = g * GVEC + k
            out_v[j, pl.ds(0, 16)] = lo
            out_v[j, pl.ds(16, 16)] = hi
            if k < NSLOT:
                fire(vec[k + NSLOT], slot)
            else:

                @pl.when(jnp.logical_not(last))
                def _():
                    fire(vec_next[k - NSLOT], slot)

        return carry

    lax.fori_loop(0, GROUPS, body, 0)
    pltpu.sync_copy(out_v, out_hbm.at[pl.ds(base, B_PER_W)])


def kernel(actionList, table):
    return _gather_kernel(actionList.astype(jnp.int32).T, table.T)


# speedup vs baseline: 1.0100x; 1.0025x over previous
import functools

import jax
import jax.numpy as jnp
from jax import lax
from jax.experimental import pallas as pl
from jax.experimental.pallas import tpu as pltpu
from jax.experimental.pallas import tpu_sc as plsc

B = 16384
D = 32
L = 50
LANES = 128
BAND = (L - 1) // 8 * 8
BROW = (L - 1) - BAND

_info = plsc.get_sparse_core_info()
_NC = _info.num_cores
_NS = _info.num_subcores
NW = _NC * _NS
B_PER_W = B // NW
NSLOT = 8
GVEC = 16
GROUPS = B_PER_W // GVEC

_mesh = plsc.VectorSubcoreMesh(core_axis_name="c", subcore_axis_name="s")


@functools.partial(
    pl.kernel,
    mesh=_mesh,
    out_type=jax.ShapeDtypeStruct((B, D), jnp.float32),
    scratch_types=[
        pltpu.VMEM((L - BAND, B_PER_W), jnp.int32),
        pltpu.VMEM((NSLOT, D, LANES), jnp.float32),
        pltpu.VMEM((B_PER_W, D), jnp.float32),
        pltpu.SemaphoreType.DMA((NSLOT,)),
    ],
    compiler_params=pltpu.CompilerParams(needs_layout_passes=False),
)
def _gather_kernel(act_t_hbm, table_t_hbm, out_hbm, idx_v, stage_v, out_v, sems):
    wid = lax.axis_index("s") * _NC + lax.axis_index("c")
    base = wid * B_PER_W
    pltpu.sync_copy(
        act_t_hbm.at[pl.ds(BAND, L - BAND), pl.ds(base, B_PER_W)], idx_v
    )

    rows_lo = lax.iota(jnp.int32, 16)
    rows_hi = rows_lo + 16

    def fire(v, slot):
        col0 = pl.multiple_of((v >> 7) * LANES, LANES)
        pltpu.async_copy(
            table_t_hbm.at[:, pl.ds(col0, LANES)],
            stage_v.at[slot],
            sems.at[slot],
        )

    def vec16(g):
        return idx_v[BROW, pl.ds(g * GVEC, GVEC)]

    vec0 = vec16(0)
    for k in range(NSLOT):
        fire(vec0[k], k)

    def body(g, carry):
        vec = vec16(g)
        vec_next = vec16((g + 1) % GROUPS)
        last = g == GROUPS - 1
        for k in range(GVEC):
            slot = k % NSLOT
            pltpu.make_async_copy(
                table_t_hbm.at[:, pl.ds(0, LANES)],
                stage_v.at[slot],
                sems.at[slot],
            ).wait()
            lane = jnp.full((16,), vec[k] & 127, jnp.int32)
            lo = plsc.load_gather(stage_v.at[slot], [rows_lo, lane])
            hi = plsc.load_gather(stage_v.at[slot], [rows_hi, lane])
            j = g * GVEC + k
            out_v[j, pl.ds(0, 16)] = lo
            out_v[j, pl.ds(16, 16)] = hi
            if k < NSLOT:
                fire(vec[k + NSLOT], slot)
            else:

                @pl.when(jnp.logical_not(last))
                def _():
                    fire(vec_next[k - NSLOT], slot)

        return carry

    lax.fori_loop(0, GROUPS, body, 0)
    pltpu.sync_copy(out_v, out_hbm.at[pl.ds(base, B_PER_W)])


def kernel(actionList, table):
    return _gather_kernel(actionList.astype(jnp.int32).T, table.T)
